# Initial kernel scaffold; baseline (speedup 1.0000x reference)
import jax
import jax.numpy as jnp
from jax.experimental import pallas as pl


def kernel(z, W1, b1, W2, b2):
    raise NotImplementedError("write your pallas kernel here")


# speedup vs baseline: 1.7107x; 1.7107x over previous
import numpy as np
import jax
import jax.numpy as jnp
from jax.experimental import pallas as pl
from jax.experimental.pallas import tpu as pltpu

CLAMP = 10.0
BBLK = 512


def _made_masks(D, H):
    m_in = np.arange(D)
    m0 = np.arange(H) % (D - 1)
    mask1 = (m_in[None, :] <= m0[:, None]).astype(np.float32)
    base = (m0[None, :] < m_in[:, None]).astype(np.float32)
    mask2 = np.repeat(base, 2, axis=0).astype(np.float32)
    return mask1, mask2


def _flow_kernel(z_ref, w1t_ref, b1_ref, wmu_ref, wal_ref, b2mu_ref, b2al_ref,
                 x_ref, ld_ref, acc_ref):
    Bb, D = z_ref.shape
    H = b1_ref.shape[1]

    acc_ref[...] = jnp.broadcast_to(b1_ref[...], (Bb, H))
    x_ref[...] = jnp.zeros((Bb, D), jnp.float32)
    z = z_ref[...]
    lane_iota = jax.lax.broadcasted_iota(jnp.int32, (1, D), 1)
    ld0 = jnp.zeros((Bb, 1), jnp.float32)

    def body(i, ld):
        t = jnp.tanh(acc_ref[...])
        wmu = wmu_ref[pl.ds(i, 1), :]
        wal = wal_ref[pl.ds(i, 1), :]
        onehot = (lane_iota == i).astype(jnp.float32)
        b2m = jnp.sum(b2mu_ref[...] * onehot, axis=1, keepdims=True)
        b2a = jnp.sum(b2al_ref[...] * onehot, axis=1, keepdims=True)
        mu = jnp.clip(jnp.sum(t * wmu, axis=1, keepdims=True) + b2m,
                      -CLAMP, CLAMP)
        al = jnp.clip(jnp.sum(t * wal, axis=1, keepdims=True) + b2a,
                      -CLAMP, CLAMP)
        z_i = jnp.sum(z * onehot, axis=1, keepdims=True)
        x_i = z_i * jnp.exp(al) + mu
        w1row = w1t_ref[pl.ds(i, 1), :]
        acc_ref[...] = acc_ref[...] + x_i * w1row
        x_ref[...] = x_ref[...] + x_i * onehot
        return ld + al

    ld = jax.lax.fori_loop(0, D, body, ld0)

    x = x_ref[...]
    x_ref[...] = jnp.where(jnp.isnan(x) | jnp.isinf(x), 0.0, x)
    ld_ref[...] = jnp.where(jnp.isnan(ld) | jnp.isinf(ld), 0.0, ld)


def kernel(z, W1, b1, W2, b2):
    B, D = z.shape
    H = W1.shape[0]
    mask1, mask2 = _made_masks(D, H)
    w1t = (W1 * mask1).T
    W2m = W2 * mask2
    wmu = W2m[:D]
    wal = W2m[D:]
    b1r = b1.reshape(1, H)
    b2mu = b2[:D].reshape(1, D)
    b2al = b2[D:].reshape(1, D)

    x, ld = pl.pallas_call(
        _flow_kernel,
        grid=(B // BBLK,),
        in_specs=[
            pl.BlockSpec((BBLK, D), lambda i: (i, 0)),
            pl.BlockSpec((D, H), lambda i: (0, 0)),
            pl.BlockSpec((1, H), lambda i: (0, 0)),
            pl.BlockSpec((D, H), lambda i: (0, 0)),
            pl.BlockSpec((D, H), lambda i: (0, 0)),
            pl.BlockSpec((1, D), lambda i: (0, 0)),
            pl.BlockSpec((1, D), lambda i: (0, 0)),
        ],
        out_specs=[
            pl.BlockSpec((BBLK, D), lambda i: (i, 0)),
            pl.BlockSpec((BBLK, 1), lambda i: (i, 0)),
        ],
        out_shape=[
            jax.ShapeDtypeStruct((B, D), jnp.float32),
            jax.ShapeDtypeStruct((B, 1), jnp.float32),
        ],
        scratch_shapes=[pltpu.VMEM((BBLK, H), jnp.float32)],
        compiler_params=pltpu.CompilerParams(
            dimension_semantics=("parallel",),
        ),
    )(z, w1t, b1r, wmu, wal, b2mu, b2al)
    return x, ld.reshape(B)
